# baseline (device time: 131926 ns/iter reference)
import jax
import jax.numpy as jnp
from jax import lax
from jax.experimental import pallas as pl
from jax.experimental.pallas import tpu as pltpu

N_X = 2
K = 16


def kernel(x):
    m, n = x.shape
    half = m // 2
    c = half // K
    n_chunks = m // c

    def body(x_hbm, out_hbm, own_bf16, miss_bf16, stag, stag_sem,
             x_send, x_recv, y_send, y_recv, own_sem, miss_sem):
        my_x = lax.axis_index("x")
        my_y = lax.axis_index("y")
        my_z = lax.axis_index("z")
        x_peer = (1 - my_x, my_y, my_z)
        y_peer = (my_x, 1 - my_y, my_z)

        miss = (1 - my_x) * m
        mine = my_y * half

        barrier_sem = pltpu.get_barrier_semaphore()
        for nbr in [x_peer, y_peer]:
            pl.semaphore_signal(
                barrier_sem, inc=1, device_id=nbr,
                device_id_type=pl.DeviceIdType.MESH,
            )
        pl.semaphore_wait(barrier_sem, 2)

        other = (1 - my_y) * half

        def stage_start(off, slot):
            cp = pltpu.make_async_copy(
                x_hbm.at[pl.ds(off, c), :], stag.at[slot], stag_sem.at[slot],
            )
            cp.start()
            return cp

        x_rdmas = []
        cps = {0: stage_start(mine, 0)}
        for k in range(K):
            slot = k % 2
            if k + 1 < K:
                cps[(k + 1) % 2] = stage_start(mine + (k + 1) * c, (k + 1) % 2)
            cps[slot].wait()
            own_bf16[pl.ds(mine + k * c, c), :] = (
                stag[slot, :, :].astype(jnp.bfloat16))
            rdma = pltpu.make_async_remote_copy(
                src_ref=own_bf16.at[pl.ds(mine + k * c, c), :],
                dst_ref=miss_bf16.at[pl.ds(mine + k * c, c), :],
                send_sem=x_send.at[k],
                recv_sem=x_recv.at[k],
                device_id=x_peer,
                device_id_type=pl.DeviceIdType.MESH,
            )
            rdma.start()
            x_rdmas.append(rdma)

        own_a = pltpu.make_async_copy(
            own_bf16.at[pl.ds(mine, half), :],
            out_hbm.at[pl.ds(my_x * m + mine, half), :],
            own_sem.at[0],
        )
        own_a.start()

        y_rdmas = []
        cps = {0: stage_start(other, 0)}
        for k in range(K):
            x_rdmas[k].wait_recv()
            rdma = pltpu.make_async_remote_copy(
                src_ref=miss_bf16.at[pl.ds(mine + k * c, c), :],
                dst_ref=miss_bf16.at[pl.ds(mine + k * c, c), :],
                send_sem=y_send.at[k],
                recv_sem=y_recv.at[k],
                device_id=y_peer,
                device_id_type=pl.DeviceIdType.MESH,
            )
            rdma.start()
            y_rdmas.append(rdma)
            slot = k % 2
            if k + 1 < K:
                cps[(k + 1) % 2] = stage_start(other + (k + 1) * c, (k + 1) % 2)
            cps[slot].wait()
            own_bf16[pl.ds(other + k * c, c), :] = (
                stag[slot, :, :].astype(jnp.bfloat16))

        own_b = pltpu.make_async_copy(
            own_bf16.at[pl.ds(other, half), :],
            out_hbm.at[pl.ds(my_x * m + other, half), :],
            own_sem.at[1],
        )
        own_b.start()

        miss_a = pltpu.make_async_copy(
            miss_bf16.at[pl.ds(mine, half), :],
            out_hbm.at[pl.ds(miss + mine, half), :],
            miss_sem.at[0],
        )
        miss_a.start()

        for k in range(K):
            y_rdmas[k].wait_recv()
        miss_b = pltpu.make_async_copy(
            miss_bf16.at[pl.ds(other, half), :],
            out_hbm.at[pl.ds(miss + other, half), :],
            miss_sem.at[1],
        )
        miss_b.start()
        for k in range(K):
            x_rdmas[k].wait_send()
            y_rdmas[k].wait_send()
        own_a.wait()
        own_b.wait()
        miss_a.wait()
        miss_b.wait()

    return pl.pallas_call(
        body,
        out_shape=jax.ShapeDtypeStruct((N_X * m, n), jnp.bfloat16),
        in_specs=[pl.BlockSpec(memory_space=pl.ANY)],
        out_specs=pl.BlockSpec(memory_space=pl.ANY),
        scratch_shapes=[
            pltpu.VMEM((m, n), jnp.bfloat16),
            pltpu.VMEM((m, n), jnp.bfloat16),
            pltpu.VMEM((2, c, n), jnp.float32),
            pltpu.SemaphoreType.DMA((2,)),
            pltpu.SemaphoreType.DMA((K,)),
            pltpu.SemaphoreType.DMA((K,)),
            pltpu.SemaphoreType.DMA((K,)),
            pltpu.SemaphoreType.DMA((K,)),
            pltpu.SemaphoreType.DMA((2,)),
            pltpu.SemaphoreType.DMA((2,)),
        ],
        compiler_params=pltpu.CompilerParams(
            collective_id=0,
            vmem_limit_bytes=48 * 1024 * 1024,
        ),
    )(x)
